# baseline (speedup 1.0000x reference)
import functools

import jax
import jax.numpy as jnp
from jax import lax
from jax.experimental import pallas as pl
from jax.experimental.pallas import tpu as pltpu
from jax.experimental.pallas import tpu_sc as plsc

NIN = 26
VOCAB_P1 = 1001
VPAD = 1008
D_MODEL = 64
BATCH = 1024
SEQ = 20

_info = plsc.get_sparse_core_info()
_NC, _NS = _info.num_cores, _info.num_subcores
_NW = _NC * _NS
_DB = 8
_LH = SEQ // 2
_UNITS = NIN * (D_MODEL // _DB) * 2
_UPW = _UNITS // _NW


def _make_emb():
    mesh = plsc.VectorSubcoreMesh(core_axis_name="c", subcore_axis_name="s")

    @functools.partial(
        pl.kernel,
        mesh=mesh,
        out_type=jax.ShapeDtypeStruct(
            (SEQ, NIN, D_MODEL // 8, BATCH // 128, 8, 128), jnp.float32),
        scratch_types=[
            pltpu.VMEM((_LH, BATCH), jnp.int32),
            pltpu.VMEM((_LH, BATCH), jnp.int32),
            pltpu.VMEM((VPAD,), jnp.float32),
            pltpu.VMEM((VPAD,), jnp.float32),
            pltpu.VMEM((_LH, BATCH // 128, 128), jnp.float32),
            pltpu.VMEM((_LH, BATCH // 128, 128), jnp.float32),
            pltpu.SemaphoreType.DMA,
            pltpu.SemaphoreType.DMA,
            pltpu.SemaphoreType.DMA,
            pltpu.SemaphoreType.DMA,
        ],
        compiler_params=pltpu.CompilerParams(
            use_tc_tiling_on_sc=False, needs_layout_passes=False),
    )
    def emb(xt_hbm, tabt_hbm, out_hbm, ib0, ib1, trb0, trb1, ob0, ob1,
            isem, tsem, osem0, osem1):
        wid = lax.axis_index("s") * _NC + lax.axis_index("c")
        ibufs = (ib0, ib1)
        trbufs = (trb0, trb1)
        obufs = (ob0, ob1)
        osems = (osem0, osem1)

        def unit_params(i):
            u = wid + i * _NW
            f = u // 16
            r = u - f * 16
            db = r // 2
            lbase = (r - db * 2) * _LH
            return f, db, lbase

        def i_copy(i, buf):
            f, db, lbase = unit_params(i)
            return pltpu.make_async_copy(
                xt_hbm.at[f, pl.ds(lbase, _LH)], buf, isem)

        def t_copy(i, dd, buf):
            f, db, lbase = unit_params(i)
            return pltpu.make_async_copy(
                tabt_hbm.at[f, db * 8 + dd, pl.ds(0, VPAD)], buf, tsem)

        def o_copy(i, dd, buf, sem):
            f, db, lbase = unit_params(i)
            return pltpu.make_async_copy(
                buf, out_hbm.at[pl.ds(lbase, _LH), f, db, :, dd], sem)

        i_copy(0, ib0).start()
        t_copy(0, 0, trb0).start()

        def one_unit(i, ib, ib_next, first):
            i_copy(i, ib).wait()

            @pl.when(i + 1 < _UPW)
            def _():
                i_copy(i + 1, ib_next).start()

            for dd in range(_DB):
                trb = trbufs[dd % 2]
                ob = obufs[dd % 2]
                t_copy(i, dd, trb).wait()
                if dd + 1 < _DB:
                    t_copy(i, dd + 1, trbufs[(dd + 1) % 2]).start()
                else:
                    @pl.when(i + 1 < _UPW)
                    def _():
                        t_copy(i + 1, 0, trbufs[0]).start()
                if not (first and dd < 2):
                    o_copy(i, dd, ob, osems[dd % 2]).wait()

                def l_body(l, carry2):
                    @plsc.parallel_loop(0, BATCH // 16, unroll=16)
                    def gather_c(c):
                        ivec = ib[l, pl.ds(c * 16, 16)]
                        ob[l, c // 8, pl.ds((c % 8) * 16, 16)] = (
                            plsc.load_gather(trb, [ivec]))
                    return carry2

                lax.fori_loop(0, _LH, l_body, 0)

                o_copy(i, dd, ob, osems[dd % 2]).start()

        one_unit(0, ib0, ib1, True)

        def pair_body(p, carry):
            one_unit(2 * p + 1, ib1, ib0, False)
            one_unit(2 * p + 2, ib0, ib1, False)
            return carry

        lax.fori_loop(0, (_UPW - 1) // 2, pair_body, 0)
        for j in range(2):
            o_copy(0, j, obufs[j], osems[j]).wait()

    return emb


def kernel(x, tables):
    xt = jnp.transpose(x.astype(jnp.int32), (2, 1, 0))
    tabt = jnp.pad(jnp.transpose(tables, (0, 2, 1)),
                   ((0, 0), (0, 0), (0, VPAD - VOCAB_P1)))
    out_t = _make_emb()(xt, tabt)
    out_p = jnp.transpose(out_t, (3, 5, 0, 1, 2, 4))
    return out_p.reshape(BATCH, SEQ, NIN, D_MODEL)


# speedup vs baseline: 1.0015x; 1.0015x over previous
import functools

import jax
import jax.numpy as jnp
from jax import lax
from jax.experimental import pallas as pl
from jax.experimental.pallas import tpu as pltpu
from jax.experimental.pallas import tpu_sc as plsc

NIN = 26
VOCAB_P1 = 1001
VPAD = 1008
D_MODEL = 64
BATCH = 1024
SEQ = 20

_info = plsc.get_sparse_core_info()
_NC, _NS = _info.num_cores, _info.num_subcores
_NW = _NC * _NS
_DB = 8
_LH = SEQ // 2
_UNITS = NIN * (D_MODEL // _DB) * 2
_UPW = _UNITS // _NW


def _make_emb():
    mesh = plsc.VectorSubcoreMesh(core_axis_name="c", subcore_axis_name="s")

    @functools.partial(
        pl.kernel,
        mesh=mesh,
        out_type=jax.ShapeDtypeStruct(
            (SEQ, NIN, D_MODEL // 8, BATCH // 128, 8, 128), jnp.float32),
        scratch_types=[
            pltpu.VMEM((_LH, BATCH), jnp.int32),
            pltpu.VMEM((_LH, BATCH), jnp.int32),
            pltpu.VMEM((VPAD,), jnp.float32),
            pltpu.VMEM((VPAD,), jnp.float32),
            pltpu.VMEM((_LH, BATCH // 128, 128), jnp.float32),
            pltpu.VMEM((_LH, BATCH // 128, 128), jnp.float32),
            pltpu.SemaphoreType.DMA,
            pltpu.SemaphoreType.DMA,
            pltpu.SemaphoreType.DMA,
            pltpu.SemaphoreType.DMA,
        ],
        compiler_params=pltpu.CompilerParams(
            use_tc_tiling_on_sc=False, needs_layout_passes=False),
    )
    def emb(xt_hbm, tabt_hbm, out_hbm, ib0, ib1, trb0, trb1, ob0, ob1,
            isem, tsem, osem0, osem1):
        wid = lax.axis_index("s") * _NC + lax.axis_index("c")
        ibufs = (ib0, ib1)
        trbufs = (trb0, trb1)
        obufs = (ob0, ob1)
        osems = (osem0, osem1)

        def unit_params(i):
            u = wid + i * _NW
            f = u // 16
            r = u - f * 16
            db = r // 2
            lbase = (r - db * 2) * _LH
            return f, db, lbase

        def i_copy(i, buf):
            f, db, lbase = unit_params(i)
            return pltpu.make_async_copy(
                xt_hbm.at[f, pl.ds(lbase, _LH)], buf, isem)

        def t_copy(i, dd, buf):
            f, db, lbase = unit_params(i)
            return pltpu.make_async_copy(
                tabt_hbm.at[f, db * 8 + dd, pl.ds(0, VPAD)], buf, tsem)

        def o_copy(i, dd, buf, sem):
            f, db, lbase = unit_params(i)
            return pltpu.make_async_copy(
                buf, out_hbm.at[pl.ds(lbase, _LH), f, db, :, dd], sem)

        i_copy(0, ib0).start()
        t_copy(0, 0, trb0).start()

        def one_unit(i, ib, ib_next, first):
            i_copy(i, ib).wait()

            @pl.when(i + 1 < _UPW)
            def _():
                i_copy(i + 1, ib_next).start()

            for dd in range(_DB):
                trb = trbufs[dd % 2]
                ob = obufs[dd % 2]
                t_copy(i, dd, trb).wait()
                if dd + 1 < _DB:
                    t_copy(i, dd + 1, trbufs[(dd + 1) % 2]).start()
                else:
                    @pl.when(i + 1 < _UPW)
                    def _():
                        t_copy(i + 1, 0, trbufs[0]).start()
                if not (first and dd < 2):
                    o_copy(i, dd, ob, osems[dd % 2]).wait()

                def l_body(l, carry2):
                    @plsc.parallel_loop(0, BATCH // 16, unroll=8)
                    def gather_c(c):
                        ivec = ib[l, pl.ds(c * 16, 16)]
                        ob[l, c // 8, pl.ds((c % 8) * 16, 16)] = (
                            plsc.load_gather(trb, [ivec]))
                    return carry2

                lax.fori_loop(0, _LH, l_body, 0)

                o_copy(i, dd, ob, osems[dd % 2]).start()

        one_unit(0, ib0, ib1, True)

        def pair_body(p, carry):
            one_unit(2 * p + 1, ib1, ib0, False)
            one_unit(2 * p + 2, ib0, ib1, False)
            return carry

        lax.fori_loop(0, (_UPW - 1) // 2, pair_body, 0)
        for j in range(2):
            o_copy(0, j, obufs[j], osems[j]).wait()

    return emb


def kernel(x, tables):
    xt = jnp.transpose(x.astype(jnp.int32), (2, 1, 0))
    tabt = jnp.pad(jnp.transpose(tables, (0, 2, 1)),
                   ((0, 0), (0, 0), (0, VPAD - VOCAB_P1)))
    out_t = _make_emb()(xt, tabt)
    out_p = jnp.transpose(out_t, (3, 5, 0, 1, 2, 4))
    return out_p.reshape(BATCH, SEQ, NIN, D_MODEL)


# speedup vs baseline: 1.0891x; 1.0874x over previous
import functools

import jax
import jax.numpy as jnp
from jax import lax
from jax.experimental import pallas as pl
from jax.experimental.pallas import tpu as pltpu
from jax.experimental.pallas import tpu_sc as plsc

NIN = 26
VOCAB_P1 = 1001
D_MODEL = 64
BATCH = 1024
SEQ = 20
TAB_ELEMS = NIN * D_MODEL * VOCAB_P1
TBLK = 8 * VOCAB_P1 + 8

_info = plsc.get_sparse_core_info()
_NC, _NS = _info.num_cores, _info.num_subcores
_NW = _NC * _NS
_DB = 8
_LH = SEQ // 2
_UNITS = NIN * (D_MODEL // _DB) * 2
_UPW = _UNITS // _NW


def _make_emb():
    mesh = plsc.VectorSubcoreMesh(core_axis_name="c", subcore_axis_name="s")

    @functools.partial(
        pl.kernel,
        mesh=mesh,
        out_type=jax.ShapeDtypeStruct(
            (SEQ, NIN, D_MODEL // 8, BATCH // 128, 8, 128), jnp.float32),
        scratch_types=[
            pltpu.VMEM((_LH, BATCH), jnp.int32),
            pltpu.VMEM((_LH, BATCH), jnp.int32),
            pltpu.VMEM((TBLK,), jnp.float32),
            pltpu.VMEM((TBLK,), jnp.float32),
            pltpu.VMEM((_LH, BATCH // 128, 128), jnp.float32),
            pltpu.VMEM((_LH, BATCH // 128, 128), jnp.float32),
            pltpu.SemaphoreType.DMA,
            pltpu.SemaphoreType.DMA,
            pltpu.SemaphoreType.DMA,
            pltpu.SemaphoreType.DMA,
        ],
        compiler_params=pltpu.CompilerParams(
            use_tc_tiling_on_sc=False, needs_layout_passes=False),
    )
    def emb(xt_hbm, tab_hbm, out_hbm, ib0, ib1, trb0, trb1, ob0, ob1,
            isem, tsem, osem0, osem1):
        wid = lax.axis_index("s") * _NC + lax.axis_index("c")
        obufs = (ob0, ob1)
        osems = (osem0, osem1)

        def unit_params(i):
            u = wid + i * _NW
            f = u // 16
            r = u - f * 16
            db = r // 2
            lbase = (r - db * 2) * _LH
            return f, db, lbase

        def tab_base(i):
            f, db, lbase = unit_params(i)
            orig = (f * D_MODEL + db * 8) * VOCAB_P1
            base = orig - orig % 8
            base = jnp.where(base + TBLK > TAB_ELEMS, base - 8, base)
            return pl.multiple_of(base, 8), orig - base

        def i_copy(i, buf):
            f, db, lbase = unit_params(i)
            return pltpu.make_async_copy(
                xt_hbm.at[f, pl.ds(lbase, _LH)], buf, isem)

        def t_copy(i, buf):
            base, _ = tab_base(i)
            return pltpu.make_async_copy(
                tab_hbm.at[pl.ds(base, TBLK)], buf, tsem)

        def o_copy(i, dd, buf, sem):
            f, db, lbase = unit_params(i)
            return pltpu.make_async_copy(
                buf, out_hbm.at[pl.ds(lbase, _LH), f, db, :, dd], sem)

        i_copy(0, ib0).start()
        t_copy(0, trb0).start()

        def one_unit(i, ib, ib_next, trb, trb_next, first):
            i_copy(i, ib).wait()
            t_copy(i, trb).wait()

            @pl.when(i + 1 < _UPW)
            def _():
                i_copy(i + 1, ib_next).start()
                t_copy(i + 1, trb_next).start()

            _, delta = tab_base(i)
            for dd in range(_DB):
                ob = obufs[dd % 2]
                if not (first and dd < 2):
                    o_copy(i, dd, ob, osems[dd % 2]).wait()

                off = delta + dd * VOCAB_P1

                def l_body(l, carry2):
                    @plsc.parallel_loop(0, BATCH // 16, unroll=8)
                    def gather_c(c):
                        ivec = ib[l, pl.ds(c * 16, 16)] + off
                        ob[l, c // 8, pl.ds((c % 8) * 16, 16)] = (
                            plsc.load_gather(trb, [ivec]))
                    return carry2

                lax.fori_loop(0, _LH, l_body, 0)

                o_copy(i, dd, ob, osems[dd % 2]).start()

        one_unit(0, ib0, ib1, trb0, trb1, True)

        def pair_body(p, carry):
            one_unit(2 * p + 1, ib1, ib0, trb1, trb0, False)
            one_unit(2 * p + 2, ib0, ib1, trb0, trb1, False)
            return carry

        lax.fori_loop(0, (_UPW - 1) // 2, pair_body, 0)
        for j in range(2):
            o_copy(0, j, obufs[j], osems[j]).wait()

    return emb


def kernel(x, tables):
    xt = jnp.transpose(x.astype(jnp.int32), (2, 1, 0))
    tab = jnp.transpose(tables, (0, 2, 1)).reshape(TAB_ELEMS)
    out_t = _make_emb()(xt, tab)
    out_p = jnp.transpose(out_t, (3, 5, 0, 1, 2, 4))
    return out_p.reshape(BATCH, SEQ, NIN, D_MODEL)


# speedup vs baseline: 1.4595x; 1.3401x over previous
import functools

import jax
import jax.numpy as jnp
from jax import lax
from jax.experimental import pallas as pl
from jax.experimental.pallas import tpu as pltpu
from jax.experimental.pallas import tpu_sc as plsc

NIN = 26
VOCAB_P1 = 1001
D_MODEL = 64
BATCH = 1024
SEQ = 20
TAB_ELEMS = NIN * D_MODEL * VOCAB_P1
TBLK = 8 * VOCAB_P1 + 8

_info = plsc.get_sparse_core_info()
_NC, _NS = _info.num_cores, _info.num_subcores
_NW = _NC * _NS
_DB = 8
_LH = SEQ // 2
_UNITS = NIN * (D_MODEL // _DB) * 2
_UPW = _UNITS // _NW


def _make_emb():
    mesh = plsc.VectorSubcoreMesh(core_axis_name="c", subcore_axis_name="s")

    @functools.partial(
        pl.kernel,
        mesh=mesh,
        out_type=jax.ShapeDtypeStruct(
            (SEQ, NIN, D_MODEL // 8, BATCH // 128, 8, 128), jnp.float32),
        scratch_types=[
            pltpu.VMEM((_LH, BATCH), jnp.int32),
            pltpu.VMEM((_LH, BATCH), jnp.int32),
            pltpu.VMEM((TBLK,), jnp.float32),
            pltpu.VMEM((TBLK,), jnp.float32),
            pltpu.VMEM((_LH, BATCH // 128, 4, 128), jnp.float32),
            pltpu.VMEM((_LH, BATCH // 128, 4, 128), jnp.float32),
            pltpu.SemaphoreType.DMA,
            pltpu.SemaphoreType.DMA,
            pltpu.SemaphoreType.DMA,
            pltpu.SemaphoreType.DMA,
        ],
        compiler_params=pltpu.CompilerParams(
            use_tc_tiling_on_sc=False, needs_layout_passes=False),
    )
    def emb(xt_hbm, tab_hbm, out_hbm, ib0, ib1, trb0, trb1, ob0, ob1,
            isem, tsem, osem0, osem1):
        wid = lax.axis_index("s") * _NC + lax.axis_index("c")
        obufs = (ob0, ob1)
        osems = (osem0, osem1)

        def unit_params(i):
            u = wid + i * _NW
            f = u // 16
            r = u - f * 16
            db = r // 2
            lbase = (r - db * 2) * _LH
            return f, db, lbase

        def tab_base(i):
            f, db, lbase = unit_params(i)
            orig = (f * D_MODEL + db * 8) * VOCAB_P1
            base = orig - orig % 8
            base = jnp.where(base + TBLK > TAB_ELEMS, base - 8, base)
            return pl.multiple_of(base, 8), orig - base

        def i_copy(i, buf):
            f, db, lbase = unit_params(i)
            return pltpu.make_async_copy(
                xt_hbm.at[f, pl.ds(lbase, _LH)], buf, isem)

        def t_copy(i, buf):
            base, _ = tab_base(i)
            return pltpu.make_async_copy(
                tab_hbm.at[pl.ds(base, TBLK)], buf, tsem)

        def o_copy(i, dq, buf, sem):
            f, db, lbase = unit_params(i)
            return pltpu.make_async_copy(
                buf,
                out_hbm.at[pl.ds(lbase, _LH), f, db, :, pl.ds(dq * 4, 4)],
                sem)

        i_copy(0, ib0).start()
        t_copy(0, trb0).start()

        def one_unit(i, ib, ib_next, trb, trb_next, first):
            i_copy(i, ib).wait()
            t_copy(i, trb).wait()

            @pl.when(i + 1 < _UPW)
            def _():
                i_copy(i + 1, ib_next).start()
                t_copy(i + 1, trb_next).start()

            _, delta = tab_base(i)
            for dq in range(_DB // 4):
                ob = obufs[dq]
                if not first:
                    o_copy(i, dq, ob, osems[dq]).wait()

                off = delta + dq * 4 * VOCAB_P1

                def l_body(l, carry2):
                    @plsc.parallel_loop(0, BATCH // 16, unroll=8)
                    def gather_c(c):
                        ivec = ib[l, pl.ds(c * 16, 16)] + off
                        for s in range(4):
                            ob[l, c // 8, s, pl.ds((c % 8) * 16, 16)] = (
                                plsc.load_gather(trb, [ivec + s * VOCAB_P1]))
                    return carry2

                lax.fori_loop(0, _LH, l_body, 0)

                o_copy(i, dq, ob, osems[dq]).start()

        one_unit(0, ib0, ib1, trb0, trb1, True)

        def pair_body(p, carry):
            one_unit(2 * p + 1, ib1, ib0, trb1, trb0, False)
            one_unit(2 * p + 2, ib0, ib1, trb0, trb1, False)
            return carry

        lax.fori_loop(0, (_UPW - 1) // 2, pair_body, 0)
        for j in range(2):
            o_copy(0, j, obufs[j], osems[j]).wait()

    return emb


def kernel(x, tables):
    xt = jnp.transpose(x.astype(jnp.int32), (2, 1, 0))
    tab = jnp.transpose(tables, (0, 2, 1)).reshape(TAB_ELEMS)
    out_t = _make_emb()(xt, tab)
    out_p = jnp.transpose(out_t, (3, 5, 0, 1, 2, 4))
    return out_p.reshape(BATCH, SEQ, NIN, D_MODEL)
